# baseline (speedup 1.0000x reference)
import jax
import jax.numpy as jnp
from jax.experimental import pallas as pl
from jax.experimental.pallas import tpu as pltpu

EPS = 1e-05

LANE_CHUNK = 1024

IN_SCALE = 1.0000001
OUT_SCALE = 1.0000001


def _block_stats(ref, lo):
    C = ref.shape[1]
    w = slice(lo, lo + LANE_CHUNK)
    v = ref[0, 0:8, w]
    s = v
    q = v * v
    for k in range(1, C // 8):
        v = ref[0, 8 * k : 8 * k + 8, w]
        s = s + v
        q = q + v * v
    ssum = jnp.sum(s, axis=0, keepdims=True)
    ssumsq = jnp.sum(q, axis=0, keepdims=True)
    return ssum, ssumsq


def _spe_kernel(ord_ref, src_ref, xs_ref, xn_ref, out_ref):
    C = xn_ref.shape[1]
    HW = xn_ref.shape[2]

    for lo in range(0, HW, LANE_CHUNK):
        w = slice(lo, lo + LANE_CHUNK)
        sum_n, sumsq_n = _block_stats(xn_ref, lo)
        sum_s, sumsq_s = _block_stats(xs_ref, lo)

        mean_n = sum_n * (1.0 / C)
        var_n = (sumsq_n - sum_n * mean_n) * (1.0 / (C - 1))
        mean_s = sum_s * (1.0 / C)
        var_s = (sumsq_s - sum_s * mean_s) * (1.0 / (C - 1))

        ratio = jnp.sqrt((var_n + EPS) / (var_s + EPS))
        offset = mean_n - mean_s * ratio

        r_adj = ratio * (1.0 / (IN_SCALE * OUT_SCALE))
        o_adj = offset * (1.0 / (IN_SCALE * OUT_SCALE))

        for k in range(C // 8):
            sl = slice(8 * k, 8 * k + 8)
            out_ref[0, sl, w] = xs_ref[0, sl, w] * r_adj + o_adj


def kernel(x, idx_swap):
    N, C, H, W = x.shape
    HW = H * W
    xv = x.reshape(N, C, HW) * jnp.float32(IN_SCALE)

    order = jnp.argsort(idx_swap).astype(jnp.int32)
    src = jnp.take(idx_swap, order)

    grid_spec = pltpu.PrefetchScalarGridSpec(
        num_scalar_prefetch=2,
        grid=(N,),
        in_specs=[
            pl.BlockSpec((1, C, HW), lambda i, o, s: (s[i], 0, 0)),
            pl.BlockSpec((1, C, HW), lambda i, o, s: (o[i], 0, 0)),
        ],
        out_specs=pl.BlockSpec((1, C, HW), lambda i, o, s: (o[i], 0, 0)),
    )

    out = pl.pallas_call(
        _spe_kernel,
        grid_spec=grid_spec,
        out_shape=jax.ShapeDtypeStruct((N, C, HW), jnp.float32),
    )(order, src, xv, xv)
    return out.reshape(N, C, H, W) * jnp.float32(OUT_SCALE)


# speedup vs baseline: 1.4233x; 1.4233x over previous
import jax
import jax.numpy as jnp
from jax.experimental import pallas as pl
from jax.experimental.pallas import tpu as pltpu

EPS = 1e-05

LANE_CHUNK = 1024


def _block_stats(ref, lo):
    C = ref.shape[1]
    w = slice(lo, lo + LANE_CHUNK)
    v = ref[0, 0:8, w]
    s = v
    q = v * v
    for k in range(1, C // 8):
        v = ref[0, 8 * k : 8 * k + 8, w]
        s = s + v
        q = q + v * v
    ssum = jnp.sum(s, axis=0, keepdims=True)
    ssumsq = jnp.sum(q, axis=0, keepdims=True)
    return ssum, ssumsq


def _spe_kernel(ord_ref, src_ref, xs_ref, xn_ref, out_ref):
    C = xn_ref.shape[1]
    HW = xn_ref.shape[2]

    for lo in range(0, HW, LANE_CHUNK):
        w = slice(lo, lo + LANE_CHUNK)
        sum_n, sumsq_n = _block_stats(xn_ref, lo)
        sum_s, sumsq_s = _block_stats(xs_ref, lo)

        mean_n = sum_n * (1.0 / C)
        var_n = (sumsq_n - sum_n * mean_n) * (1.0 / (C - 1))
        mean_s = sum_s * (1.0 / C)
        var_s = (sumsq_s - sum_s * mean_s) * (1.0 / (C - 1))

        ratio = jnp.sqrt((var_n + EPS) / (var_s + EPS))
        offset = mean_n - mean_s * ratio

        for k in range(C // 8):
            sl = slice(8 * k, 8 * k + 8)
            out_ref[0, sl, w] = xs_ref[0, sl, w] * ratio + offset


def kernel(x, idx_swap):
    N, C, H, W = x.shape
    HW = H * W
    xv = x.reshape(N, C, HW)

    order = jnp.argsort(idx_swap).astype(jnp.int32)
    src = jnp.take(idx_swap, order)

    grid_spec = pltpu.PrefetchScalarGridSpec(
        num_scalar_prefetch=2,
        grid=(N,),
        in_specs=[
            pl.BlockSpec((1, C, HW), lambda i, o, s: (s[i], 0, 0)),
            pl.BlockSpec((1, C, HW), lambda i, o, s: (o[i], 0, 0)),
        ],
        out_specs=pl.BlockSpec((1, C, HW), lambda i, o, s: (o[i], 0, 0)),
    )

    out = pl.pallas_call(
        _spe_kernel,
        grid_spec=grid_spec,
        out_shape=jax.ShapeDtypeStruct((N, C, HW), jnp.float32),
    )(order, src, xv, xv)
    return out.reshape(N, C, H, W)
